# baseline (device time: 62936 ns/iter reference)
import jax
import jax.numpy as jnp
from jax import lax
from jax.experimental import pallas as pl
from jax.experimental.pallas import tpu as pltpu

N_Y = 4


def kernel(x):
    m_per, n = x.shape
    n_out = n // N_Y
    m_out = m_per * N_Y
    m_q = m_per // 4
    m_h = m_q // 2

    def body(
        x_ref, out_ref,
        comm_y, zin, xin, ddh,
        ys, yr, zs1, zr1, xs1, xr1, zs2, zr2, xs2, xr2,
    ):
        mx = lax.axis_index("x")
        my = lax.axis_index("y")
        mz = lax.axis_index("z")
        px = 1 - mx
        zb = lax.rem(mz, 2)
        pz = mz + 1 - 2 * zb

        barrier_sem = pltpu.get_barrier_semaphore()
        for d in range(1, N_Y):
            pl.semaphore_signal(
                barrier_sem, inc=1,
                device_id=(mx, lax.rem(my + d, N_Y), mz),
                device_id_type=pl.DeviceIdType.MESH,
            )
        for dev in ((mx, my, pz), (px, my, mz)):
            pl.semaphore_signal(
                barrier_sem, inc=1, device_id=dev,
                device_id_type=pl.DeviceIdType.MESH,
            )
        pl.semaphore_wait(barrier_sem, N_Y + 1)

        for d in range(N_Y - 1, 0, -1):
            tgt = lax.rem(my + d, N_Y)
            pltpu.make_async_remote_copy(
                src_ref=x_ref.at[
                    pl.ds(mx * 2 * m_q + zb * m_q, m_q),
                    pl.ds(tgt * n_out, n_out),
                ],
                dst_ref=comm_y.at[d - 1],
                send_sem=ys.at[d - 1],
                recv_sem=yr.at[d - 1],
                device_id=(mx, tgt, mz),
                device_id_type=pl.DeviceIdType.MESH,
            ).start()

        out_ref[pl.ds(my * m_per, m_per), :] = (
            x_ref[:, pl.ds(my * n_out, n_out)]
        )

        def wait_recv(buf, slot, rsem):
            pltpu.make_async_remote_copy(
                src_ref=buf.at[slot], dst_ref=buf.at[slot],
                send_sem=rsem.at[slot], recv_sem=rsem.at[slot],
                device_id=(mx, my, mz),
                device_id_type=pl.DeviceIdType.MESH,
            ).wait_recv()

        def wait_half(buf, slot, rsem):
            pltpu.make_async_remote_copy(
                src_ref=buf.at[slot, pl.ds(0, m_h), :],
                dst_ref=buf.at[slot, pl.ds(0, m_h), :],
                send_sem=rsem.at[slot], recv_sem=rsem.at[slot],
                device_id=(mx, my, mz),
                device_id_type=pl.DeviceIdType.MESH,
            ).wait_recv()

        def send(src_ref, dst_ref, ssem, rsem, slot, dev):
            pltpu.make_async_remote_copy(
                src_ref=src_ref, dst_ref=dst_ref,
                send_sem=ssem.at[slot], recv_sem=rsem.at[slot],
                device_id=dev, device_id_type=pl.DeviceIdType.MESH,
            ).start()

        def rows(base, xbit, zbit):
            return base + xbit * 2 * m_q + zbit * m_q

        for d in range(1, N_Y):
            k = d - 1
            base = lax.rem(my - d + N_Y, N_Y) * m_per
            wait_recv(comm_y, k, yr)
            send(comm_y.at[k], zin.at[k], zs1, zr1, k, (mx, my, pz))
            send(comm_y.at[k], xin.at[k], xs1, xr1, k, (px, my, mz))
            out_ref[pl.ds(rows(base, mx, zb), m_q), :] = comm_y[k]
            wait_recv(zin, k, zr1)
            send(zin.at[k, pl.ds(m_h, m_h), :], ddh.at[k, pl.ds(m_h, m_h), :],
                 xs2, xr2, k, (px, my, mz))
            out_ref[pl.ds(rows(base, mx, 1 - zb), m_q), :] = zin[k]
            wait_recv(xin, k, xr1)
            send(xin.at[k, pl.ds(0, m_h), :], ddh.at[k, pl.ds(0, m_h), :],
                 zs2, zr2, k, (mx, my, pz))
            out_ref[pl.ds(rows(base, px, zb), m_q), :] = xin[k]

        for d in range(1, N_Y):
            k = d - 1
            base = lax.rem(my - d + N_Y, N_Y) * m_per
            wait_half(ddh, k, zr2)
            wait_half(ddh, k, xr2)
            out_ref[pl.ds(rows(base, px, 1 - zb), m_q), :] = ddh[k]

        for sems in (ys, zs1, xs1):
            for k in range(N_Y - 1):
                pltpu.make_async_remote_copy(
                    src_ref=comm_y.at[k], dst_ref=comm_y.at[k],
                    send_sem=sems.at[k], recv_sem=sems.at[k],
                    device_id=(mx, my, mz),
                    device_id_type=pl.DeviceIdType.MESH,
                ).wait_send()
        for sems in (zs2, xs2):
            for k in range(N_Y - 1):
                pltpu.make_async_remote_copy(
                    src_ref=ddh.at[k, pl.ds(0, m_h), :],
                    dst_ref=ddh.at[k, pl.ds(0, m_h), :],
                    send_sem=sems.at[k], recv_sem=sems.at[k],
                    device_id=(mx, my, mz),
                    device_id_type=pl.DeviceIdType.MESH,
                ).wait_send()

    out_shape = jax.ShapeDtypeStruct((m_out, n_out), x.dtype)
    semq = pltpu.SemaphoreType.DMA((N_Y - 1,))
    bufq = pltpu.VMEM((N_Y - 1, m_q, n_out), x.dtype)
    bufh = pltpu.VMEM((N_Y - 1, m_h, n_out), x.dtype)
    return pl.pallas_call(
        body,
        out_shape=out_shape,
        in_specs=[pl.BlockSpec(memory_space=pltpu.VMEM)],
        out_specs=pl.BlockSpec(memory_space=pltpu.VMEM),
        scratch_shapes=[
            bufq,
            bufq,
            bufq,
            bufq,
            semq, semq,
            semq, semq,
            semq, semq,
            semq, semq,
            semq, semq,
        ],
        compiler_params=pltpu.CompilerParams(collective_id=0),
    )(x)


# device time: 52297 ns/iter; 1.2034x vs baseline; 1.2034x over previous
import jax
import jax.numpy as jnp
from jax import lax
from jax.experimental import pallas as pl
from jax.experimental.pallas import tpu as pltpu

N_Y = 4


def kernel(x):
    m_per, n = x.shape
    n_out = n // N_Y
    m_out = m_per * N_Y
    m_q = m_per // 4
    m_h = m_q // 2

    def body(
        x_ref, out_ref,
        comm_y, zin, xin, ddh,
        ys, yr, zs1, zr1, xs1, xr1, zs2, zr2, xs2, xr2,
    ):
        mx = lax.axis_index("x")
        my = lax.axis_index("y")
        mz = lax.axis_index("z")
        px = 1 - mx
        zb = lax.rem(mz, 2)
        pz = mz + 1 - 2 * zb

        barrier_sem = pltpu.get_barrier_semaphore()
        for d in range(1, N_Y):
            pl.semaphore_signal(
                barrier_sem, inc=1,
                device_id=(mx, lax.rem(my + d, N_Y), mz),
                device_id_type=pl.DeviceIdType.MESH,
            )
        for dev in ((mx, my, pz), (px, my, mz)):
            pl.semaphore_signal(
                barrier_sem, inc=1, device_id=dev,
                device_id_type=pl.DeviceIdType.MESH,
            )
        pl.semaphore_wait(barrier_sem, N_Y + 1)

        for d in range(1, N_Y):
            tgt = lax.rem(my + d, N_Y)
            pltpu.make_async_remote_copy(
                src_ref=x_ref.at[
                    pl.ds(mx * 2 * m_q + zb * m_q, m_q),
                    pl.ds(tgt * n_out, n_out),
                ],
                dst_ref=comm_y.at[d - 1],
                send_sem=ys.at[d - 1],
                recv_sem=yr.at[d - 1],
                device_id=(mx, tgt, mz),
                device_id_type=pl.DeviceIdType.MESH,
            ).start()

        out_ref[pl.ds(my * m_per, m_per), :] = (
            x_ref[:, pl.ds(my * n_out, n_out)]
        )

        def wait_recv(buf, slot, rsem):
            pltpu.make_async_remote_copy(
                src_ref=buf.at[slot], dst_ref=buf.at[slot],
                send_sem=rsem.at[slot], recv_sem=rsem.at[slot],
                device_id=(mx, my, mz),
                device_id_type=pl.DeviceIdType.MESH,
            ).wait_recv()

        def wait_half(buf, slot, rsem):
            pltpu.make_async_remote_copy(
                src_ref=buf.at[slot, pl.ds(0, m_h), :],
                dst_ref=buf.at[slot, pl.ds(0, m_h), :],
                send_sem=rsem.at[slot], recv_sem=rsem.at[slot],
                device_id=(mx, my, mz),
                device_id_type=pl.DeviceIdType.MESH,
            ).wait_recv()

        def send(src_ref, dst_ref, ssem, rsem, slot, dev):
            pltpu.make_async_remote_copy(
                src_ref=src_ref, dst_ref=dst_ref,
                send_sem=ssem.at[slot], recv_sem=rsem.at[slot],
                device_id=dev, device_id_type=pl.DeviceIdType.MESH,
            ).start()

        def rows(base, xbit, zbit):
            return base + xbit * 2 * m_q + zbit * m_q

        for d in range(1, N_Y):
            k = d - 1
            base = lax.rem(my - d + N_Y, N_Y) * m_per
            wait_recv(comm_y, k, yr)
            send(comm_y.at[k], zin.at[k], zs1, zr1, k, (mx, my, pz))
            send(comm_y.at[k], xin.at[k], xs1, xr1, k, (px, my, mz))
            out_ref[pl.ds(rows(base, mx, zb), m_q), :] = comm_y[k]
            wait_recv(zin, k, zr1)
            send(zin.at[k, pl.ds(m_h, m_h), :], ddh.at[k, pl.ds(m_h, m_h), :],
                 xs2, xr2, k, (px, my, mz))
            out_ref[pl.ds(rows(base, mx, 1 - zb), m_q), :] = zin[k]
            wait_recv(xin, k, xr1)
            send(xin.at[k, pl.ds(0, m_h), :], ddh.at[k, pl.ds(0, m_h), :],
                 zs2, zr2, k, (mx, my, pz))
            out_ref[pl.ds(rows(base, px, zb), m_q), :] = xin[k]

        for d in range(1, N_Y):
            k = d - 1
            base = lax.rem(my - d + N_Y, N_Y) * m_per
            wait_half(ddh, k, zr2)
            wait_half(ddh, k, xr2)
            out_ref[pl.ds(rows(base, px, 1 - zb), m_q), :] = ddh[k]

        for sems in (ys, zs1, xs1):
            for k in range(N_Y - 1):
                pltpu.make_async_remote_copy(
                    src_ref=comm_y.at[k], dst_ref=comm_y.at[k],
                    send_sem=sems.at[k], recv_sem=sems.at[k],
                    device_id=(mx, my, mz),
                    device_id_type=pl.DeviceIdType.MESH,
                ).wait_send()
        for sems in (zs2, xs2):
            for k in range(N_Y - 1):
                pltpu.make_async_remote_copy(
                    src_ref=ddh.at[k, pl.ds(0, m_h), :],
                    dst_ref=ddh.at[k, pl.ds(0, m_h), :],
                    send_sem=sems.at[k], recv_sem=sems.at[k],
                    device_id=(mx, my, mz),
                    device_id_type=pl.DeviceIdType.MESH,
                ).wait_send()

    out_shape = jax.ShapeDtypeStruct((m_out, n_out), x.dtype)
    semq = pltpu.SemaphoreType.DMA((N_Y - 1,))
    bufq = pltpu.VMEM((N_Y - 1, m_q, n_out), x.dtype)
    bufh = pltpu.VMEM((N_Y - 1, m_h, n_out), x.dtype)
    return pl.pallas_call(
        body,
        out_shape=out_shape,
        in_specs=[pl.BlockSpec(memory_space=pltpu.VMEM)],
        out_specs=pl.BlockSpec(memory_space=pltpu.VMEM),
        scratch_shapes=[
            bufq,
            bufq,
            bufq,
            bufq,
            semq, semq,
            semq, semq,
            semq, semq,
            semq, semq,
            semq, semq,
        ],
        compiler_params=pltpu.CompilerParams(collective_id=0),
    )(x)


# device time: 52277 ns/iter; 1.2039x vs baseline; 1.0004x over previous
import jax
import jax.numpy as jnp
from jax import lax
from jax.experimental import pallas as pl
from jax.experimental.pallas import tpu as pltpu

N_Y = 4


def kernel(x):
    m_per, n = x.shape
    n_out = n // N_Y
    m_out = m_per * N_Y
    m_q = m_per // 4
    m_h = m_q // 2

    def body(
        x_ref, out_ref,
        comm_y, zin, xin, ddh,
        ys, yr, zs1, zr1, xs1, xr1, zs2, zr2, xs2, xr2,
    ):
        mx = lax.axis_index("x")
        my = lax.axis_index("y")
        mz = lax.axis_index("z")
        px = 1 - mx
        zb = lax.rem(mz, 2)
        pz = mz + 1 - 2 * zb

        barrier_sem = pltpu.get_barrier_semaphore()
        for d in range(1, N_Y):
            pl.semaphore_signal(
                barrier_sem, inc=1,
                device_id=(mx, lax.rem(my + d, N_Y), mz),
                device_id_type=pl.DeviceIdType.MESH,
            )
        for dev in ((mx, my, pz), (px, my, mz)):
            pl.semaphore_signal(
                barrier_sem, inc=1, device_id=dev,
                device_id_type=pl.DeviceIdType.MESH,
            )
        pl.semaphore_wait(barrier_sem, N_Y + 1)

        for d in range(1, N_Y):
            tgt = lax.rem(my + d, N_Y)
            pltpu.make_async_remote_copy(
                src_ref=x_ref.at[
                    pl.ds(mx * 2 * m_q + zb * m_q, m_q),
                    pl.ds(tgt * n_out, n_out),
                ],
                dst_ref=comm_y.at[d - 1],
                send_sem=ys.at[d - 1],
                recv_sem=yr.at[d - 1],
                device_id=(mx, tgt, mz),
                device_id_type=pl.DeviceIdType.MESH,
            ).start()

        out_ref[pl.ds(my * m_per, m_per), :] = (
            x_ref[:, pl.ds(my * n_out, n_out)]
        )

        def wait_recv(buf, slot, rsem):
            pltpu.make_async_remote_copy(
                src_ref=buf.at[slot], dst_ref=buf.at[slot],
                send_sem=rsem.at[slot], recv_sem=rsem.at[slot],
                device_id=(mx, my, mz),
                device_id_type=pl.DeviceIdType.MESH,
            ).wait_recv()

        def wait_half(buf, slot, rsem):
            pltpu.make_async_remote_copy(
                src_ref=buf.at[slot, pl.ds(0, m_h), :],
                dst_ref=buf.at[slot, pl.ds(0, m_h), :],
                send_sem=rsem.at[slot], recv_sem=rsem.at[slot],
                device_id=(mx, my, mz),
                device_id_type=pl.DeviceIdType.MESH,
            ).wait_recv()

        def send(src_ref, dst_ref, ssem, rsem, slot, dev):
            pltpu.make_async_remote_copy(
                src_ref=src_ref, dst_ref=dst_ref,
                send_sem=ssem.at[slot], recv_sem=rsem.at[slot],
                device_id=dev, device_id_type=pl.DeviceIdType.MESH,
            ).start()

        def rows(base, xbit, zbit):
            return base + xbit * 2 * m_q + zbit * m_q

        for d in range(1, N_Y):
            k = d - 1
            base = lax.rem(my - d + N_Y, N_Y) * m_per
            wait_recv(comm_y, k, yr)
            send(comm_y.at[k], zin.at[k], zs1, zr1, k, (mx, my, pz))
            send(comm_y.at[k], xin.at[k], xs1, xr1, k, (px, my, mz))
            out_ref[pl.ds(rows(base, mx, zb), m_q), :] = comm_y[k]
            wait_recv(zin, k, zr1)
            send(zin.at[k, pl.ds(m_h, m_h), :], ddh.at[k, pl.ds(m_h, m_h), :],
                 xs2, xr2, k, (px, my, mz))
            out_ref[pl.ds(rows(base, mx, 1 - zb), m_q), :] = zin[k]
            wait_recv(xin, k, xr1)
            send(xin.at[k, pl.ds(0, m_h), :], ddh.at[k, pl.ds(0, m_h), :],
                 zs2, zr2, k, (mx, my, pz))
            out_ref[pl.ds(rows(base, px, zb), m_q), :] = xin[k]

        for d in range(1, N_Y):
            k = d - 1
            base = lax.rem(my - d + N_Y, N_Y) * m_per
            wait_half(ddh, k, zr2)
            wait_half(ddh, k, xr2)
            out_ref[pl.ds(rows(base, px, 1 - zb), m_q), :] = ddh[k]

        for sems in (ys, zs1, xs1):
            for k in range(N_Y - 1):
                pltpu.make_async_remote_copy(
                    src_ref=comm_y.at[k], dst_ref=comm_y.at[k],
                    send_sem=sems.at[k], recv_sem=sems.at[k],
                    device_id=(mx, my, mz),
                    device_id_type=pl.DeviceIdType.MESH,
                ).wait_send()
        for sems in (zs2, xs2):
            for k in range(N_Y - 1):
                pltpu.make_async_remote_copy(
                    src_ref=ddh.at[k, pl.ds(0, m_h), :],
                    dst_ref=ddh.at[k, pl.ds(0, m_h), :],
                    send_sem=sems.at[k], recv_sem=sems.at[k],
                    device_id=(mx, my, mz),
                    device_id_type=pl.DeviceIdType.MESH,
                ).wait_send()

    out_shape = jax.ShapeDtypeStruct((m_out, n_out), x.dtype)
    semq = pltpu.SemaphoreType.DMA((N_Y - 1,))
    bufq = pltpu.VMEM((N_Y - 1, m_q, n_out), x.dtype)
    return pl.pallas_call(
        body,
        out_shape=out_shape,
        in_specs=[pl.BlockSpec(memory_space=pltpu.VMEM)],
        out_specs=pl.BlockSpec(memory_space=pltpu.VMEM),
        scratch_shapes=[
            bufq,
            bufq,
            bufq,
            bufq,
            semq, semq,
            semq, semq,
            semq, semq,
            semq, semq,
            semq, semq,
        ],
        compiler_params=pltpu.CompilerParams(collective_id=0),
    )(x)
